# baseline (speedup 1.0000x reference)
import functools

import jax
import jax.numpy as jnp
from jax import lax
from jax.experimental import pallas as pl
from jax.experimental.pallas import tpu as pltpu
from jax.experimental.pallas import tpu_sc as plsc

_L = 128
_D = 64
_H = 50
_NT = 128


@functools.lru_cache(maxsize=None)
def _build(n_rows: int):
    info = plsc.get_sparse_core_info()
    nc, ns = info.num_cores, info.num_subcores
    nw = nc * ns

    n_slabs = _H * _NT
    per_w = n_slabs // nw

    mesh = plsc.VectorSubcoreMesh(core_axis_name="c", subcore_axis_name="s")

    @functools.partial(
        pl.kernel,
        mesh=mesh,
        compiler_params=pltpu.CompilerParams(
            use_tc_tiling_on_sc=True, needs_layout_passes=False
        ),
        out_type=jax.ShapeDtypeStruct((_H, _D, _NT * _L), jnp.float32),
        scratch_types=[
            pltpu.VMEM((per_w, _L), jnp.int32),
            pltpu.VMEM((2, 8, _L), jnp.int32),
            pltpu.VMEM((2, _L, _L + 4), jnp.float32),
            pltpu.VMEM((2, _D, _L), jnp.float32),
            pltpu.SemaphoreType.DMA,
            pltpu.SemaphoreType.DMA,
            pltpu.SemaphoreType.DMA,
            pltpu.SemaphoreType.DMA,
        ],
    )
    def gather_kernel(xr_hbm, tab_hbm, out_hbm, idx_v, id2_v, rec_v, slab_v, g0, g1, s0, s1):
        gsems = (g0, g1)
        ssems = (s0, s1)
        wid = lax.axis_index("s") * nc + lax.axis_index("c")
        s_base = wid * per_w
        pltpu.sync_copy(xr_hbm.at[pl.ds(s_base, per_w)], idx_v)

        iotas = [lax.iota(jnp.int32, 16) + 16 * seg for seg in range(8)]

        def prep_ids(sl, slot):
            for seg in range(8):
                v = idx_v[sl, pl.ds(seg * 16, 16)]
                id2_v[slot, 0, pl.ds(seg * 16, 16)] = v >> 1
                id2_v[slot, 1, pl.ds(seg * 16, 16)] = (v & 1) << 6

        def gather_desc(slot):
            return pltpu.make_async_copy(
                tab_hbm.at[id2_v.at[slot, 0]],
                rec_v.at[slot].at[:, pl.ds(0, _L)],
                gsems[slot],
            )

        def store_desc(s, slot):
            h = s // _NT
            t = s % _NT
            return pltpu.make_async_copy(
                slab_v.at[slot],
                out_hbm.at[h].at[:, pl.ds(t * _L, _L)],
                ssems[slot],
            )

        def transpose(slot):
            offs = [id2_v[slot, 1, pl.ds(seg * 16, 16)] for seg in range(8)]

            @plsc.parallel_loop(0, _D, step=1, unroll=8)
            def dbody(d):
                for seg in range(8):
                    vals = plsc.load_gather(
                        rec_v.at[slot], [iotas[seg], offs[seg] + d]
                    )
                    slab_v[slot, d, pl.ds(seg * 16, 16)] = vals

        prep_ids(0, 0)
        gather_desc(0).start()

        def body(s2, carry):
            for slot in (0, 1):
                sl = s2 * 2 + slot
                o = 1 - slot
                gather_desc(slot).wait()

                @pl.when(sl + 1 < per_w)
                def _():
                    prep_ids(sl + 1, o)

                    @pl.when(sl >= 1)
                    def _():
                        store_desc(s_base + sl - 1, o).wait()

                    gather_desc(o).start()

                transpose(slot)
                store_desc(s_base + sl, slot).start()
            return carry

        lax.fori_loop(0, per_w // 2, body, 0)
        store_desc(s_base + per_w - 2, 0).wait()
        store_desc(s_base + per_w - 1, 1).wait()

    return gather_kernel


def kernel(x, table):
    b, h = x.shape
    v, d = table.shape

    xr = jnp.transpose(x.astype(jnp.int32)).reshape(h * (b // _L), _L)
    tab2 = table.reshape(v // 2, 2 * d)

    gather_kernel = _build(xr.shape[0])
    out_t = gather_kernel(xr, tab2)
    return jnp.transpose(out_t, (2, 0, 1))


# speedup vs baseline: 1.1573x; 1.1573x over previous
import functools

import jax
import jax.numpy as jnp
from jax import lax
from jax.experimental import pallas as pl
from jax.experimental.pallas import tpu as pltpu
from jax.experimental.pallas import tpu_sc as plsc

_L = 128
_D = 64
_H = 50
_NT = 128


@functools.lru_cache(maxsize=None)
def _build(n_rows: int):
    info = plsc.get_sparse_core_info()
    nc, ns = info.num_cores, info.num_subcores
    nw = nc * ns

    n_slabs = _H * _NT
    per_w = n_slabs // nw

    mesh = plsc.VectorSubcoreMesh(core_axis_name="c", subcore_axis_name="s")

    @functools.partial(
        pl.kernel,
        mesh=mesh,
        compiler_params=pltpu.CompilerParams(
            use_tc_tiling_on_sc=True, needs_layout_passes=False
        ),
        out_type=jax.ShapeDtypeStruct((_H, _D, _NT * _L), jnp.float32),
        scratch_types=[
            pltpu.VMEM((per_w, _L), jnp.int32),
            pltpu.VMEM((2, 8, _L), jnp.int32),
            pltpu.VMEM((2, _L, _L), jnp.float32),
            pltpu.VMEM((2, _D, _L), jnp.float32),
            pltpu.SemaphoreType.DMA,
            pltpu.SemaphoreType.DMA,
            pltpu.SemaphoreType.DMA,
            pltpu.SemaphoreType.DMA,
        ],
    )
    def gather_kernel(xr_hbm, tab_hbm, out_hbm, idx_v, id2_v, rec_v, slab_v, g0, g1, s0, s1):
        gsems = (g0, g1)
        ssems = (s0, s1)
        wid = lax.axis_index("s") * nc + lax.axis_index("c")
        s_base = wid * per_w
        pltpu.sync_copy(xr_hbm.at[pl.ds(s_base, per_w)], idx_v)

        iotas = [lax.iota(jnp.int32, 16) + 16 * seg for seg in range(8)]

        def prep_ids(sl, slot):
            for seg in range(8):
                v = idx_v[sl, pl.ds(seg * 16, 16)]
                id2_v[slot, 0, pl.ds(seg * 16, 16)] = v >> 1
                id2_v[slot, 1, pl.ds(seg * 16, 16)] = (v & 1) << 6

        def gather_desc(slot):
            return pltpu.make_async_copy(
                tab_hbm.at[id2_v.at[slot, 0]],
                rec_v.at[slot],
                gsems[slot],
            )

        def store_desc(s, slot):
            h = s // _NT
            t = s % _NT
            return pltpu.make_async_copy(
                slab_v.at[slot],
                out_hbm.at[h].at[:, pl.ds(t * _L, _L)],
                ssems[slot],
            )

        def transpose(slot):
            offs = [id2_v[slot, 1, pl.ds(seg * 16, 16)] for seg in range(8)]

            @plsc.parallel_loop(0, _D, step=1, unroll=8)
            def dbody(d):
                for seg in range(8):
                    vals = plsc.load_gather(
                        rec_v.at[slot], [iotas[seg], offs[seg] + d]
                    )
                    slab_v[slot, d, pl.ds(seg * 16, 16)] = vals

        prep_ids(0, 0)
        gather_desc(0).start()

        def body(s2, carry):
            for slot in (0, 1):
                sl = s2 * 2 + slot
                o = 1 - slot
                gather_desc(slot).wait()

                @pl.when(sl + 1 < per_w)
                def _():
                    prep_ids(sl + 1, o)

                    @pl.when(sl >= 1)
                    def _():
                        store_desc(s_base + sl - 1, o).wait()

                    gather_desc(o).start()

                transpose(slot)
                store_desc(s_base + sl, slot).start()
            return carry

        lax.fori_loop(0, per_w // 2, body, 0)
        store_desc(s_base + per_w - 2, 0).wait()
        store_desc(s_base + per_w - 1, 1).wait()

    return gather_kernel


def kernel(x, table):
    b, h = x.shape
    v, d = table.shape

    xr = jnp.transpose(x.astype(jnp.int32)).reshape(h * (b // _L), _L)
    tab2 = table.reshape(v // 2, 2 * d)

    gather_kernel = _build(xr.shape[0])
    out_t = gather_kernel(xr, tab2)
    return jnp.transpose(out_t, (2, 0, 1))


# speedup vs baseline: 1.1639x; 1.0057x over previous
import functools

import jax
import jax.numpy as jnp
from jax import lax
from jax.experimental import pallas as pl
from jax.experimental.pallas import tpu as pltpu
from jax.experimental.pallas import tpu_sc as plsc

_L = 128
_D = 64
_H = 50
_NT = 128


@functools.lru_cache(maxsize=None)
def _build(n_rows: int):
    info = plsc.get_sparse_core_info()
    nc, ns = info.num_cores, info.num_subcores
    nw = nc * ns

    n_slabs = _H * _NT
    per_w = n_slabs // nw

    mesh = plsc.VectorSubcoreMesh(core_axis_name="c", subcore_axis_name="s")

    @functools.partial(
        pl.kernel,
        mesh=mesh,
        compiler_params=pltpu.CompilerParams(
            use_tc_tiling_on_sc=True, needs_layout_passes=False
        ),
        out_type=jax.ShapeDtypeStruct((_H, _D, _NT * _L), jnp.float32),
        scratch_types=[
            pltpu.VMEM((per_w, _L), jnp.int32),
            pltpu.VMEM((2, 8, _L), jnp.int32),
            pltpu.VMEM((2, _L, _L), jnp.float32),
            pltpu.VMEM((2, _D, _L), jnp.float32),
            pltpu.SemaphoreType.DMA,
            pltpu.SemaphoreType.DMA,
            pltpu.SemaphoreType.DMA,
            pltpu.SemaphoreType.DMA,
        ],
    )
    def gather_kernel(xr_hbm, tab_hbm, out_hbm, idx_v, id2_v, rec_v, slab_v, g0, g1, s0, s1):
        gsems = (g0, g1)
        ssems = (s0, s1)
        wid = lax.axis_index("s") * nc + lax.axis_index("c")
        s_base = wid * per_w
        pltpu.sync_copy(xr_hbm.at[pl.ds(s_base, per_w)], idx_v)

        iotas = [lax.iota(jnp.int32, 16) + 16 * seg for seg in range(8)]

        def prep_ids(sl, slot):
            for seg in range(8):
                v = idx_v[sl, pl.ds(seg * 16, 16)]
                id2_v[slot, 0, pl.ds(seg * 16, 16)] = v >> 1
                id2_v[slot, 1, pl.ds(seg * 16, 16)] = (v & 1) << 6

        def gather_desc(slot):
            return pltpu.make_async_copy(
                tab_hbm.at[id2_v.at[slot, 0]],
                rec_v.at[slot],
                gsems[slot],
            )

        def store_desc(s, slot):
            h = s // _NT
            t = s % _NT
            return pltpu.make_async_copy(
                slab_v.at[slot],
                out_hbm.at[h].at[:, pl.ds(t * _L, _L)],
                ssems[slot],
            )

        def transpose(slot):
            offs = [id2_v[slot, 1, pl.ds(seg * 16, 16)] for seg in range(8)]

            @plsc.parallel_loop(0, _D, step=1, unroll=16)
            def dbody(d):
                for seg in range(8):
                    vals = plsc.load_gather(
                        rec_v.at[slot], [iotas[seg], offs[seg] + d]
                    )
                    slab_v[slot, d, pl.ds(seg * 16, 16)] = vals

        prep_ids(0, 0)
        gather_desc(0).start()

        def body(s2, carry):
            for slot in (0, 1):
                sl = s2 * 2 + slot
                o = 1 - slot
                gather_desc(slot).wait()

                @pl.when(sl + 1 < per_w)
                def _():
                    prep_ids(sl + 1, o)

                    @pl.when(sl >= 1)
                    def _():
                        store_desc(s_base + sl - 1, o).wait()

                    gather_desc(o).start()

                transpose(slot)
                store_desc(s_base + sl, slot).start()
            return carry

        lax.fori_loop(0, per_w // 2, body, 0)
        store_desc(s_base + per_w - 2, 0).wait()
        store_desc(s_base + per_w - 1, 1).wait()

    return gather_kernel


def kernel(x, table):
    b, h = x.shape
    v, d = table.shape

    xr = jnp.transpose(x.astype(jnp.int32)).reshape(h * (b // _L), _L)
    tab2 = table.reshape(v // 2, 2 * d)

    gather_kernel = _build(xr.shape[0])
    out_t = gather_kernel(xr, tab2)
    return jnp.transpose(out_t, (2, 0, 1))
